# baseline (speedup 1.0000x reference)
import functools

import jax
import jax.numpy as jnp
from jax.experimental import pallas as pl

_dot = functools.partial(jnp.dot, preferred_element_type=jnp.float32)

B = 65536
H = 64
NL = 3
NS = 2
NR = 4
NE = NS + NR
BLK = 4096
W1C = NE * H + 2 * NR


def _moe_kernel(xt_ref, in_Wt_ref, in_b_ref, Wcat1_ref, bcat1_ref,
                W2bd_ref, b2pair_ref, sel_ref,
                out_Wt_ref, out_b_ref, u_ref):
    f32 = jnp.float32
    xt = xt_ref[...]
    h = jnp.tanh(_dot(xt, in_Wt_ref[...]) + in_b_ref[...])
    for l in range(NL):
        pre = _dot(h, Wcat1_ref[l]) + bcat1_ref[l]
        a = jnp.tanh(pre[:, :NE * H])
        lp = pre[:, NE * H:]
        lT = jnp.transpose(lp)
        r = [lT[i:i + 1, :] for i in range(NR)]
        m = jnp.maximum(jnp.maximum(r[0], r[1]), jnp.maximum(r[2], r[3]))
        e = [jnp.exp(r[i] - m) for i in range(NR)]
        s = ((e[0] + e[1]) + e[2]) + e[3]
        rw = [e[i] / s for i in range(NR)]
        wrow = []
        for i in range(NR):
            rank = jnp.zeros_like(rw[i])
            for j in range(NR):
                if j == i:
                    continue
                beats = (rw[j] > rw[i]) if j > i else (rw[j] >= rw[i])
                rank = rank + beats.astype(f32)
            wrow.append(rw[i] * (rank < 2.0).astype(f32))
        wT = jnp.concatenate(wrow + [jnp.zeros_like(lT[:NR])], axis=0)
        w4 = jnp.transpose(wT)
        p_sh = _dot(a[:, 0:2 * H], W2bd_ref[l, 0])
        p_r01 = _dot(a[:, 2 * H:4 * H], W2bd_ref[l, 1])
        p_r23 = _dot(a[:, 4 * H:6 * H], W2bd_ref[l, 2])
        wboth = _dot(w4, sel_ref[...])
        psb = p_sh + b2pair_ref[l, 0]
        rp01 = wboth[:, :2 * H] * (p_r01 + b2pair_ref[l, 1])
        rp23 = wboth[:, 2 * H:] * (p_r23 + b2pair_ref[l, 2])
        shared = psb[:, :H] + psb[:, H:]
        routed = ((rp01[:, :H] + rp01[:, H:]) + rp23[:, :H]) + rp23[:, H:]
        h = jnp.tanh(h + shared + routed)
    u_ref[...] = _dot(h, out_Wt_ref[...]) + out_b_ref[...]


@jax.jit
def kernel(x, t, in_W, in_b, sh_W1, sh_b1, sh_W2, sh_b2,
           rt_W1, rt_b1, rt_W2, rt_b2, rtr_W, rtr_b, out_W, out_b):
    f32 = jnp.float32
    xt = jnp.concatenate([x, t], axis=1)
    W1all = jnp.concatenate([sh_W1, rt_W1], axis=1)
    b1all = jnp.concatenate([sh_b1, rt_b1], axis=1)
    W2all = jnp.concatenate([sh_W2, rt_W2], axis=1)
    Wc = jnp.transpose(W1all, (0, 3, 1, 2)).reshape(NL, H, NE * H)
    rtr_Wt = jnp.transpose(rtr_W, (0, 2, 1))
    Wcat1 = jnp.concatenate(
        [Wc, rtr_Wt, jnp.zeros((NL, H, NR), f32)], axis=2)
    bcat1 = jnp.concatenate(
        [b1all.reshape(NL, NE * H), rtr_b, jnp.zeros((NL, NR), f32)], axis=1)
    W2t = jnp.transpose(W2all, (0, 1, 3, 2))
    z = jnp.zeros((NL, 3, H, H), f32)
    top = jnp.concatenate([W2t[:, 0::2], z], axis=3)
    bot = jnp.concatenate([z, W2t[:, 1::2]], axis=3)
    W2bd = jnp.concatenate([top, bot], axis=2)
    b2pair = jnp.concatenate([sh_b2, rt_b2], axis=1).reshape(NL, 3, 2 * H)
    sel = jnp.concatenate(
        [jnp.repeat(jnp.eye(NR, dtype=f32), H, axis=1),
         jnp.zeros((NR, NR * H), f32)], axis=0)
    in_Wt = in_W.T
    out_Wt = out_W.T
    in_b2 = in_b.reshape(1, H)
    out_b2 = out_b.reshape(1, 1)

    grid = (B // BLK,)
    full = lambda *s: pl.BlockSpec(s, lambda i: (0,) * len(s))
    u = pl.pallas_call(
        _moe_kernel,
        grid=grid,
        in_specs=[
            pl.BlockSpec((BLK, 2), lambda i: (i, 0)),
            full(2, H),
            full(1, H),
            full(NL, H, W1C),
            full(NL, W1C),
            full(NL, 3, 2 * H, 2 * H),
            full(NL, 3, 2 * H),
            full(2 * NR, NR * H),
            full(H, 1),
            full(1, 1),
        ],
        out_specs=pl.BlockSpec((BLK, 1), lambda i: (i, 0)),
        out_shape=jax.ShapeDtypeStruct((B, 1), jnp.float32),
    )(xt, in_Wt, in_b2, Wcat1, bcat1, W2bd, b2pair, sel,
      out_Wt, out_b2)
    return u
